# baseline (speedup 1.0000x reference)
import functools

import jax
import jax.numpy as jnp
from jax import lax
from jax.experimental import pallas as pl
from jax.experimental.pallas import tpu as pltpu
from jax.experimental.pallas import tpu_sc as plsc

_M = 10
_K = 256
_SUB = 30
_SUBP = 32
_NC = 2
_NS = 16
_NW = _NC * _NS
_C = 128
_LANES = 16


@functools.lru_cache(maxsize=None)
def _build(num_tokens: int, vec_elems: int):
    tokens_per_worker = num_tokens // _NW
    num_chunks = tokens_per_worker // _C
    mesh = plsc.VectorSubcoreMesh(
        core_axis_name="c", subcore_axis_name="s",
        num_cores=_NC, num_subcores=_NS)

    @functools.partial(
        pl.kernel,
        out_type=jax.ShapeDtypeStruct((num_tokens, _M * _SUBP), jnp.float32),
        mesh=mesh,
        scratch_types=[
            pltpu.VMEM((tokens_per_worker,), jnp.int32),
            pltpu.VMEM((_M, _C), jnp.int32),
            pltpu.VMEM((_M, _C), jnp.int32),
            pltpu.VMEM((_M, _C), jnp.int32),
            pltpu.VMEM((_M, _C, _SUBP), jnp.float32),
            pltpu.VMEM_SHARED((vec_elems,), jnp.int32),
            pltpu.VMEM_SHARED((_M * _K, _SUBP), jnp.float32),
            pltpu.SemaphoreType.DMA,
            pltpu.SemaphoreType.DMA,
            pltpu.SemaphoreType.DMA,
        ],
        compiler_params=pltpu.CompilerParams(use_tc_tiling_on_sc=False),
    )
    def pq_lookup(ids_hbm, vec_hbm, cw_hbm, out_hbm,
                  ids_v, idx1_v, codes_v, idx2_v, rows_v, vec_sh, cw_sh,
                  sem_codes, sem_rows, sem_w):
        wid = lax.axis_index("s") * _NC + lax.axis_index("c")
        base0 = wid * tokens_per_worker

        @pl.when(lax.axis_index("s") == 0)
        def _load_tables():
            pltpu.sync_copy(vec_hbm, vec_sh)
            pltpu.sync_copy(cw_hbm, cw_sh)
        plsc.subcore_barrier()

        pltpu.sync_copy(ids_hbm.at[pl.ds(base0, tokens_per_worker)], ids_v)

        def chunk_body(c, carry):
            tok = base0 + c * _C
            loc = c * _C

            for j in range(_M):
                def mk1(k, carry2, j=j):
                    s = pl.ds(k * _LANES, _LANES)
                    idx1_v[j, s] = ids_v[pl.ds(loc + k * _LANES, _LANES)] * _M + j
                    return carry2
                lax.fori_loop(0, _C // _LANES, mk1, 0, unroll=True)

            code_copies = [
                pltpu.async_copy(
                    vec_sh.at[idx1_v.at[j]], codes_v.at[j], sem_codes)
                for j in range(_M)]
            for cp in code_copies:
                cp.wait()

            for j in range(_M):
                def mk2(k, carry2, j=j):
                    s = pl.ds(k * _LANES, _LANES)
                    idx2_v[j, s] = codes_v[j, s] + j * _K
                    return carry2
                lax.fori_loop(0, _C // _LANES, mk2, 0, unroll=True)

            @pl.when(c > 0)
            def _drain_prev():
                for j in range(_M):
                    pltpu.make_async_copy(
                        rows_v.at[j],
                        out_hbm.at[pl.ds(tok - _C, _C),
                                   pl.ds(j * _SUBP, _SUBP)],
                        sem_w).wait()

            row_copies = [
                pltpu.async_copy(
                    cw_sh.at[idx2_v.at[j]], rows_v.at[j], sem_rows)
                for j in range(_M)]
            for cp in row_copies:
                cp.wait()

            for j in range(_M):
                pltpu.async_copy(
                    rows_v.at[j],
                    out_hbm.at[pl.ds(tok, _C), pl.ds(j * _SUBP, _SUBP)],
                    sem_w)
            return carry

        lax.fori_loop(0, num_chunks, chunk_body, 0)

        last = base0 + (num_chunks - 1) * _C
        for j in range(_M):
            pltpu.make_async_copy(
                rows_v.at[j],
                out_hbm.at[pl.ds(last, _C), pl.ds(j * _SUBP, _SUBP)],
                sem_w).wait()

    return pq_lookup


@functools.lru_cache(maxsize=None)
def _build_strip(b: int, l: int):
    bb = 64
    grid = b // bb
    row_elems = _M * _SUBP
    rows128 = bb * l * row_elems // 128
    pairs = bb * l // 2

    def body(x_ref, o_ref):
        x5 = x_ref[...].reshape(pairs, 5, 128)
        par = []
        for p in range(2):
            segs = []
            for j in range(_M):
                off = p * row_elems + j * _SUBP
                r, c = off // 128, off % 128
                segs.append(x5[:, r, c:c + _SUB])
            par.append(jnp.concatenate(segs, axis=1).reshape(pairs, 1, -1))
        y = jnp.concatenate(par, axis=1)
        o_ref[...] = y.reshape(bb, l, _M * _SUB)

    return pl.pallas_call(
        body,
        grid=(grid,),
        in_specs=[pl.BlockSpec((rows128, 128), lambda i: (i, 0))],
        out_specs=pl.BlockSpec((bb, l, _M * _SUB), lambda i: (i, 0, 0)),
        out_shape=jax.ShapeDtypeStruct((b, l, _M * _SUB), jnp.float32),
    )


def kernel(input_ids, codewords, vectors):
    b, l = input_ids.shape
    n = b * l
    ids = input_ids.reshape(n)
    vec = vectors.reshape(-1)
    cw = jnp.pad(codewords.reshape(_M * _K, _SUB),
                 ((0, 0), (0, _SUBP - _SUB)))
    out = _build(n, vec.shape[0])(ids, vec, cw)
    out128 = out.reshape(n * _M * _SUBP // 128, 128)
    return _build_strip(b, l)(out128)


# speedup vs baseline: 1.0559x; 1.0559x over previous
import functools

import jax
import jax.numpy as jnp
from jax import lax
from jax.experimental import pallas as pl
from jax.experimental.pallas import tpu as pltpu
from jax.experimental.pallas import tpu_sc as plsc

_M = 10
_K = 256
_SUB = 30
_SUBP = 32
_NC = 2
_NS = 16
_NW = _NC * _NS
_C = 128
_LANES = 16


@functools.lru_cache(maxsize=None)
def _build(num_tokens: int, vec_elems: int):
    tokens_per_worker = num_tokens // _NW
    num_chunks = tokens_per_worker // _C
    mesh = plsc.VectorSubcoreMesh(
        core_axis_name="c", subcore_axis_name="s",
        num_cores=_NC, num_subcores=_NS)

    @functools.partial(
        pl.kernel,
        out_type=jax.ShapeDtypeStruct((num_tokens, _M * _SUBP), jnp.float32),
        mesh=mesh,
        scratch_types=[
            pltpu.VMEM((tokens_per_worker,), jnp.int32),
            pltpu.VMEM((_M, _C), jnp.int32),
            pltpu.VMEM((_M, _C), jnp.int32),
            pltpu.VMEM((_M, _C), jnp.int32),
            pltpu.VMEM((_M, _C, _SUBP), jnp.float32),
            pltpu.VMEM_SHARED((vec_elems,), jnp.int32),
            pltpu.VMEM_SHARED((_M * _K, _SUBP), jnp.float32),
            pltpu.SemaphoreType.DMA,
            pltpu.SemaphoreType.DMA,
            pltpu.SemaphoreType.DMA,
        ],
        compiler_params=pltpu.CompilerParams(use_tc_tiling_on_sc=False),
    )
    def pq_lookup(ids_hbm, vec_hbm, cw_hbm, out_hbm,
                  ids_v, idx1_v, codes_v, idx2_v, rows_v, vec_sh, cw_sh,
                  sem_codes, sem_rows, sem_w):
        wid = lax.axis_index("s") * _NC + lax.axis_index("c")
        base0 = wid * tokens_per_worker

        @pl.when(lax.axis_index("s") == 0)
        def _load_tables():
            pltpu.sync_copy(vec_hbm, vec_sh)
            pltpu.sync_copy(cw_hbm, cw_sh)
        plsc.subcore_barrier()

        pltpu.sync_copy(ids_hbm.at[pl.ds(base0, tokens_per_worker)], ids_v)

        def chunk_body(c, carry):
            tok = base0 + c * _C
            loc = c * _C

            for j in range(_M):
                def mk1(k, carry2, j=j):
                    s = pl.ds(k * _LANES, _LANES)
                    idx1_v[j, s] = ids_v[pl.ds(loc + k * _LANES, _LANES)] * _M + j
                    return carry2
                lax.fori_loop(0, _C // _LANES, mk1, 0, unroll=True)

            code_copies = [
                pltpu.async_copy(
                    vec_sh.at[idx1_v.at[j]], codes_v.at[j], sem_codes)
                for j in range(_M)]
            for cp in code_copies:
                cp.wait()

            for j in range(_M):
                def mk2(k, carry2, j=j):
                    s = pl.ds(k * _LANES, _LANES)
                    idx2_v[j, s] = codes_v[j, s] + j * _K
                    return carry2
                lax.fori_loop(0, _C // _LANES, mk2, 0, unroll=True)

            @pl.when(c > 0)
            def _drain_prev():
                for j in range(_M):
                    pltpu.make_async_copy(
                        rows_v.at[j],
                        out_hbm.at[pl.ds(tok - _C, _C),
                                   pl.ds(j * _SUBP, _SUBP)],
                        sem_w).wait()

            row_copies = [
                pltpu.async_copy(
                    cw_sh.at[idx2_v.at[j]], rows_v.at[j], sem_rows)
                for j in range(_M)]
            for cp in row_copies:
                cp.wait()

            for j in range(_M):
                pltpu.async_copy(
                    rows_v.at[j],
                    out_hbm.at[pl.ds(tok, _C), pl.ds(j * _SUBP, _SUBP)],
                    sem_w)
            return carry

        lax.fori_loop(0, num_chunks, chunk_body, 0)

        last = base0 + (num_chunks - 1) * _C
        for j in range(_M):
            pltpu.make_async_copy(
                rows_v.at[j],
                out_hbm.at[pl.ds(last, _C), pl.ds(j * _SUBP, _SUBP)],
                sem_w).wait()

    return pq_lookup


@functools.lru_cache(maxsize=None)
def _build_strip(b: int, l: int):
    bb = 64
    grid = b // bb

    def body(x_ref, o_ref):
        x = x_ref[...]
        parts = [x[:, j * _SUBP:j * _SUBP + _SUB] for j in range(_M)]
        o_ref[...] = jnp.concatenate(parts, axis=1).reshape(bb, l, _M * _SUB)

    return pl.pallas_call(
        body,
        grid=(grid,),
        in_specs=[pl.BlockSpec((bb * l, _M * _SUBP), lambda i: (i, 0))],
        out_specs=pl.BlockSpec((bb, l, _M * _SUB), lambda i: (i, 0, 0)),
        out_shape=jax.ShapeDtypeStruct((b, l, _M * _SUB), jnp.float32),
    )


def kernel(input_ids, codewords, vectors):
    b, l = input_ids.shape
    n = b * l
    ids = input_ids.reshape(n)
    vec = vectors.reshape(-1)
    cw = jnp.pad(codewords.reshape(_M * _K, _SUB),
                 ((0, 0), (0, _SUBP - _SUB)))
    out = _build(n, vec.shape[0])(ids, vec, cw)
    return _build_strip(b, l)(out)
